# baseline (speedup 1.0000x reference)
import functools

import jax
import jax.numpy as jnp
from jax import lax
from jax.experimental import pallas as pl
from jax.experimental.pallas import tpu as pltpu
from jax.experimental.pallas import tpu_sc as plsc

DEADE = -(1 << 28)
MANT_MASK = 0x007FFFFF
ONE_BITS = 0x3F800000
LN2HI = 0.69314575195
LN2LO = 1.42860677e-06

NCHUNK = 9
BUFLEN = 160


def _gather_kernel(lp_ref, cls_ref, w_ref):
    x = lp_ref[0]
    m = jnp.max(x, axis=1, keepdims=True)
    e = jnp.exp(x - m)
    z = jnp.sum(e, axis=1, keepdims=True)
    C = x.shape[1]
    cls = cls_ref[0]
    cidx = jax.lax.broadcasted_iota(jnp.int32, (C, 128), 0)
    oh = (cidx == cls).astype(jnp.float32)
    g = jnp.dot(e, oh, preferred_element_type=jnp.float32)
    w_ref[0] = g * (1.0 / z)


def _vdecomp(p):
    bits = lax.bitcast_convert_type(p, jnp.int32)
    e = lax.shift_right_logical(bits, 23) - 127
    m = lax.bitcast_convert_type(
        lax.bitwise_or(lax.bitwise_and(bits, MANT_MASK), ONE_BITS),
        jnp.float32)
    return m, e


def _vscale(d):
    return lax.bitcast_convert_type(
        lax.shift_left(jnp.maximum(d + 127, 0), 23), jnp.float32)


def _sc_dp_build(B, T):
    mesh = plsc.VectorSubcoreMesh(core_axis_name="c", subcore_axis_name="s")

    @functools.partial(
        pl.kernel,
        mesh=mesh,
        out_type=[
            jax.ShapeDtypeStruct((B, BUFLEN), jnp.float32),
            jax.ShapeDtypeStruct((B, BUFLEN), jnp.int32),
        ],
        scratch_types=[
            pltpu.VMEM((T * 128,), jnp.float32),
            pltpu.VMEM((BUFLEN,), jnp.float32),
            pltpu.VMEM((16,), jnp.int32),
            pltpu.VMEM((BUFLEN,), jnp.float32),
            pltpu.VMEM((BUFLEN,), jnp.int32),
            pltpu.VMEM((BUFLEN,), jnp.float32),
            pltpu.VMEM((BUFLEN,), jnp.int32),
        ],
    )
    def sc_dp(w_hbm, skip_hbm, len_hbm, m_out, e_out,
              w_v, skip_v, len_v, bmA, beA, bmB, beB):
        wid = lax.axis_index("s") * 2 + lax.axis_index("c")
        pltpu.sync_copy(w_hbm.at[wid], w_v)
        pltpu.sync_copy(skip_hbm.at[wid], skip_v)
        pltpu.sync_copy(len_hbm.at[wid], len_v)

        ones = jnp.full((16,), 1.0, jnp.float32)
        deade = jnp.full((16,), DEADE, jnp.int32)
        for off in range(0, BUFLEN, 16):
            bmA[pl.ds(off, 16)] = ones
            beA[pl.ds(off, 16)] = deade
            bmB[pl.ds(off, 16)] = ones
            beB[pl.ds(off, 16)] = deade
        p0 = w_v[pl.ds(0, 16)]
        m_i, e_i = _vdecomp(p0)
        lane16 = lax.broadcasted_iota(jnp.int32, (16,), 0)
        bmA[pl.ds(2, 16)] = jnp.where(lane16 < 2, m_i, 1.0)
        beA[pl.ds(2, 16)] = jnp.where(lane16 < 2, e_i, DEADE)

        lenv = len_v[...]
        skv = [skip_v[pl.ds(16 * c, 16)] for c in range(NCHUNK)]

        def step(t, src_m, src_e, dst_m, dst_e):
            act = jnp.full((16,), t, jnp.int32) < lenv
            for c in range(NCHUNK):
                base = 16 * c
                prow = min(base, 127 - 15)
                pch = w_v[pl.ds(t * 128 + prow, 16)]
                mS = src_m[pl.ds(base + 2, 16)]
                eS = src_e[pl.ds(base + 2, 16)]
                m1 = src_m[pl.ds(base + 1, 16)]
                e1 = src_e[pl.ds(base + 1, 16)]
                m2 = src_m[pl.ds(base, 16)]
                e2 = src_e[pl.ds(base, 16)]
                e2 = jnp.where(skv[c] > 0, e2, DEADE)
                E = jnp.maximum(jnp.maximum(eS, e1), e2)
                msum = (mS * _vscale(eS - E) + m1 * _vscale(e1 - E)
                        + m2 * _vscale(e2 - E)) * pch
                bits = lax.bitcast_convert_type(msum, jnp.int32)
                eb = lax.shift_right_logical(bits, 23)
                mN = lax.bitcast_convert_type(
                    lax.bitwise_or(lax.bitwise_and(bits, MANT_MASK),
                                   ONE_BITS), jnp.float32)
                eN = E + (eb - 127)
                dst_m[pl.ds(base + 2, 16)] = jnp.where(act, mN, mS)
                dst_e[pl.ds(base + 2, 16)] = jnp.where(act, eN, eS)

        def pair(k, carry):
            step(1 + 2 * k, bmA, beA, bmB, beB)
            step(2 + 2 * k, bmB, beB, bmA, beA)
            return carry

        lax.fori_loop(0, (T - 2) // 2, pair, 0)
        step(T - 1, bmA, beA, bmB, beB)

        pltpu.sync_copy(bmB, m_out.at[wid])
        pltpu.sync_copy(beB, e_out.at[wid])

    return sc_dp


@jax.jit
def kernel(log_probs, targets, input_lengths, target_lengths):
    B, T, C = log_probs.shape
    L = targets.shape[1]
    targets = targets.astype(jnp.int32)
    input_lengths = input_lengths.astype(jnp.int32)
    target_lengths = target_lengths.astype(jnp.int32)

    lane = jnp.arange(128, dtype=jnp.int32)[None, :]
    j64 = jnp.arange(L, dtype=jnp.int32)[:, None]
    lab = (lane - 1) // 2
    is_lab = lane % 2 == 1
    sel_cur = (is_lab & (lab == j64)).astype(jnp.float32)
    sel_prv = (is_lab & (lane >= 3) & (lab - 1 == j64)).astype(jnp.float32)
    tgtf = targets.astype(jnp.float32)
    tat = jnp.dot(tgtf, sel_cur,
                  precision=jax.lax.Precision.HIGHEST)
    pat = jnp.dot(tgtf, sel_prv, precision=jax.lax.Precision.HIGHEST)
    cls = tat.astype(jnp.int32)[:, None, :]
    skip = is_lab & (lane >= 3) & (tat != pat)
    skipf = jnp.pad(skip.astype(jnp.float32),
                    ((0, 0), (0, BUFLEN - 128)))
    len16 = jnp.broadcast_to(input_lengths[:, None], (B, 16))

    w = pl.pallas_call(
        _gather_kernel,
        grid=(B,),
        in_specs=[
            pl.BlockSpec((1, T, C), lambda i: (i, 0, 0)),
            pl.BlockSpec((1, 1, 128), lambda i: (i, 0, 0)),
        ],
        out_specs=pl.BlockSpec((1, T, 128), lambda i: (i, 0, 0)),
        out_shape=jax.ShapeDtypeStruct((B, T, 128), jnp.float32),
        compiler_params=pltpu.CompilerParams(
            dimension_semantics=("arbitrary",)),
    )(log_probs, cls)

    wf = w.reshape(B, T * 128)
    m_all, e_all = _sc_dp_build(B, T)(wf, skipf, len16)

    s_last = 2 * target_lengths
    lane_b = jnp.arange(BUFLEN, dtype=jnp.int32)[None, :]
    sel1 = (lane_b == (s_last + 2)[:, None]).astype(jnp.float32)
    sel2 = (lane_b == (s_last + 1)[:, None]).astype(jnp.float32)
    eaf = e_all.astype(jnp.float32)
    m1 = jnp.sum(m_all * sel1, axis=1)
    e1 = jnp.sum(eaf * sel1, axis=1)
    m2 = jnp.sum(m_all * sel2, axis=1)
    e2 = jnp.sum(eaf * sel2, axis=1)
    E = jnp.maximum(e1, e2)
    v = m1 * jnp.exp2((e1 - E).astype(jnp.float32)) + \
        m2 * jnp.exp2((e2 - E).astype(jnp.float32))
    ef = E.astype(jnp.float32)
    return -(jnp.log(v) + ef * LN2HI + ef * LN2LO)


# speedup vs baseline: 1.0924x; 1.0924x over previous
import functools

import jax
import jax.numpy as jnp
from jax import lax
from jax.experimental import pallas as pl
from jax.experimental.pallas import tpu as pltpu
from jax.experimental.pallas import tpu_sc as plsc

DEADE = -(1 << 28)
MANT_MASK = 0x007FFFFF
ONE_BITS = 0x3F800000
LN2HI = 0.69314575195
LN2LO = 1.42860677e-06

NCHUNK = 9
BUFLEN = 160


def _gather_kernel(lp_ref, cls_ref, w_ref):
    x = lp_ref[0]
    m = jnp.max(x, axis=1, keepdims=True)
    e = jnp.exp(x - m)
    z = jnp.sum(e, axis=1, keepdims=True)
    C = x.shape[1]
    cls = cls_ref[0]
    cidx = jax.lax.broadcasted_iota(jnp.int32, (C, 128), 0)
    oh = (cidx == cls).astype(jnp.float32)
    g = jnp.dot(e, oh, preferred_element_type=jnp.float32)
    w_ref[0] = g * (1.0 / z)


def _vdecomp(p):
    bits = lax.bitcast_convert_type(p, jnp.int32)
    e = lax.shift_right_logical(bits, 23) - 127
    m = lax.bitcast_convert_type(
        lax.bitwise_or(lax.bitwise_and(bits, MANT_MASK), ONE_BITS),
        jnp.float32)
    return m, e


def _vscale(d):
    return lax.bitcast_convert_type(
        lax.shift_left(jnp.maximum(d + 127, 0), 23), jnp.float32)


def _sc_dp_build(B, T):
    mesh = plsc.VectorSubcoreMesh(core_axis_name="c", subcore_axis_name="s")

    @functools.partial(
        pl.kernel,
        mesh=mesh,
        out_type=[
            jax.ShapeDtypeStruct((B, BUFLEN), jnp.float32),
            jax.ShapeDtypeStruct((B, BUFLEN), jnp.int32),
        ],
        scratch_types=[
            pltpu.VMEM((T, 128), jnp.float32),
            pltpu.VMEM((BUFLEN,), jnp.float32),
            pltpu.VMEM((16,), jnp.int32),
            pltpu.VMEM((BUFLEN,), jnp.float32),
            pltpu.VMEM((BUFLEN,), jnp.int32),
            pltpu.VMEM((BUFLEN,), jnp.float32),
            pltpu.VMEM((BUFLEN,), jnp.int32),
        ],
    )
    def sc_dp(w_hbm, skip_hbm, len_hbm, m_out, e_out,
              w_v, skip_v, len_v, bmA, beA, bmB, beB):
        wid = lax.axis_index("s") * 2 + lax.axis_index("c")
        pltpu.sync_copy(w_hbm.at[wid], w_v)
        pltpu.sync_copy(skip_hbm.at[wid], skip_v)
        pltpu.sync_copy(len_hbm.at[wid], len_v)

        ones = jnp.full((16,), 1.0, jnp.float32)
        deade = jnp.full((16,), DEADE, jnp.int32)
        for off in range(0, BUFLEN, 16):
            bmA[pl.ds(off, 16)] = ones
            beA[pl.ds(off, 16)] = deade
            bmB[pl.ds(off, 16)] = ones
            beB[pl.ds(off, 16)] = deade
        p0 = w_v[0, pl.ds(0, 16)]
        m_i, e_i = _vdecomp(p0)
        lane16 = lax.broadcasted_iota(jnp.int32, (16,), 0)
        bmA[pl.ds(2, 16)] = jnp.where(lane16 < 2, m_i, 1.0)
        beA[pl.ds(2, 16)] = jnp.where(lane16 < 2, e_i, DEADE)

        lenv = len_v[...]
        skv = [skip_v[pl.ds(16 * c, 16)] for c in range(NCHUNK)]

        def step(t, src_m, src_e, dst_m, dst_e):
            act = jnp.full((16,), t, jnp.int32) < lenv
            for c in range(NCHUNK):
                base = 16 * c
                prow = min(base, 127 - 15)
                pch = w_v[t, pl.ds(prow, 16)]
                mS = src_m[pl.ds(base + 2, 16)]
                eS = src_e[pl.ds(base + 2, 16)]
                m1 = src_m[pl.ds(base + 1, 16)]
                e1 = src_e[pl.ds(base + 1, 16)]
                m2 = src_m[pl.ds(base, 16)]
                e2 = src_e[pl.ds(base, 16)]
                e2 = jnp.where(skv[c] > 0, e2, DEADE)
                E = jnp.maximum(jnp.maximum(eS, e1), e2)
                msum = (mS * _vscale(eS - E) + m1 * _vscale(e1 - E)
                        + m2 * _vscale(e2 - E)) * pch
                bits = lax.bitcast_convert_type(msum, jnp.int32)
                eb = lax.shift_right_logical(bits, 23)
                mN = lax.bitcast_convert_type(
                    lax.bitwise_or(lax.bitwise_and(bits, MANT_MASK),
                                   ONE_BITS), jnp.float32)
                eN = E + (eb - 127)
                dst_m[pl.ds(base + 2, 16)] = jnp.where(act, mN, mS)
                dst_e[pl.ds(base + 2, 16)] = jnp.where(act, eN, eS)

        def pair(k, carry):
            step(1 + 2 * k, bmA, beA, bmB, beB)
            step(2 + 2 * k, bmB, beB, bmA, beA)
            return carry

        lax.fori_loop(0, (T - 2) // 2, pair, 0)
        step(T - 1, bmA, beA, bmB, beB)

        pltpu.sync_copy(bmB, m_out.at[wid])
        pltpu.sync_copy(beB, e_out.at[wid])

    return sc_dp


@jax.jit
def kernel(log_probs, targets, input_lengths, target_lengths):
    B, T, C = log_probs.shape
    L = targets.shape[1]
    targets = targets.astype(jnp.int32)
    input_lengths = input_lengths.astype(jnp.int32)
    target_lengths = target_lengths.astype(jnp.int32)

    lane = jnp.arange(128, dtype=jnp.int32)[None, :]
    j64 = jnp.arange(L, dtype=jnp.int32)[:, None]
    lab = (lane - 1) // 2
    is_lab = lane % 2 == 1
    sel_cur = (is_lab & (lab == j64)).astype(jnp.float32)
    sel_prv = (is_lab & (lane >= 3) & (lab - 1 == j64)).astype(jnp.float32)
    tgtf = targets.astype(jnp.float32)
    tat = jnp.dot(tgtf, sel_cur,
                  precision=jax.lax.Precision.HIGHEST)
    pat = jnp.dot(tgtf, sel_prv, precision=jax.lax.Precision.HIGHEST)
    cls = tat.astype(jnp.int32)[:, None, :]
    skip = is_lab & (lane >= 3) & (tat != pat)
    skipf = jnp.pad(skip.astype(jnp.float32),
                    ((0, 0), (0, BUFLEN - 128)))
    len16 = jnp.broadcast_to(input_lengths[:, None], (B, 16))

    w = pl.pallas_call(
        _gather_kernel,
        grid=(B,),
        in_specs=[
            pl.BlockSpec((1, T, C), lambda i: (i, 0, 0)),
            pl.BlockSpec((1, 1, 128), lambda i: (i, 0, 0)),
        ],
        out_specs=pl.BlockSpec((1, T, 128), lambda i: (i, 0, 0)),
        out_shape=jax.ShapeDtypeStruct((B, T, 128), jnp.float32),
        compiler_params=pltpu.CompilerParams(
            dimension_semantics=("arbitrary",)),
    )(log_probs, cls)

    m_all, e_all = _sc_dp_build(B, T)(w, skipf, len16)

    s_last = 2 * target_lengths
    lane_b = jnp.arange(BUFLEN, dtype=jnp.int32)[None, :]
    sel1 = (lane_b == (s_last + 2)[:, None]).astype(jnp.float32)
    sel2 = (lane_b == (s_last + 1)[:, None]).astype(jnp.float32)
    eaf = e_all.astype(jnp.float32)
    m1 = jnp.sum(m_all * sel1, axis=1)
    e1 = jnp.sum(eaf * sel1, axis=1)
    m2 = jnp.sum(m_all * sel2, axis=1)
    e2 = jnp.sum(eaf * sel2, axis=1)
    E = jnp.maximum(e1, e2)
    v = m1 * jnp.exp2((e1 - E).astype(jnp.float32)) + \
        m2 * jnp.exp2((e2 - E).astype(jnp.float32))
    ef = E.astype(jnp.float32)
    return -(jnp.log(v) + ef * LN2HI + ef * LN2LO)
